# baseline (speedup 1.0000x reference)
import functools

import jax
import jax.numpy as jnp
from jax import lax
from jax.experimental import pallas as pl
from jax.experimental.pallas import tpu as pltpu
from jax.experimental.pallas import tpu_sc as plsc

NUM_CORES = 2
NUM_SUBCORES = 16
NUM_WORKERS = NUM_CORES * NUM_SUBCORES
LANES = 16


@jax.jit
def kernel(hyperparameters, table):
    num_emb, em_size = table.shape
    batch = hyperparameters.shape[0]
    idx = jnp.squeeze(hyperparameters, axis=1).astype(jnp.int32)
    padded = jnp.pad(table, ((0, 0), (0, 128 - em_size)))
    b_per_w = batch // NUM_WORKERS

    mesh = plsc.VectorSubcoreMesh(core_axis_name="c", subcore_axis_name="s")

    @functools.partial(
        pl.kernel,
        mesh=mesh,
        out_type=jax.ShapeDtypeStruct((em_size, batch), jnp.float32),
        compiler_params=pltpu.CompilerParams(needs_layout_passes=False),
        scratch_types=[
            pltpu.VMEM((b_per_w,), jnp.int32),
            pltpu.VMEM((b_per_w, 128), jnp.float32),
            pltpu.VMEM((em_size, b_per_w), jnp.float32),
            pltpu.SemaphoreType.DMA,
        ],
    )
    def emb_lookup(table_hbm, idx_hbm, out_hbm, idx_v, rows_v, out_v, sem):
        wid = lax.axis_index("s") * NUM_CORES + lax.axis_index("c")
        base = wid * b_per_w
        pltpu.sync_copy(idx_hbm.at[pl.ds(base, b_per_w)], idx_v)

        @pl.loop(0, b_per_w, step=LANES)
        def _(k):
            v = idx_v[pl.ds(k, LANES)]
            pltpu.async_copy(table_hbm.at[v], rows_v.at[pl.ds(k, LANES)], sem)

        pltpu.make_async_copy(
            table_hbm.at[pl.ds(0, b_per_w)], rows_v, sem
        ).wait()

        rows16 = [
            lax.iota(jnp.int32, LANES) + c for c in range(0, em_size, LANES)
        ]

        @pl.loop(0, b_per_w, step=1)
        def _(r):
            col = jnp.full((LANES,), r, jnp.int32)
            for c in range(0, em_size, LANES):
                vals = rows_v[r, pl.ds(c, LANES)]
                plsc.store_scatter(out_v, [rows16[c // LANES], col], vals)

        pltpu.sync_copy(out_v, out_hbm.at[:, pl.ds(base, b_per_w)])

    return emb_lookup(padded, idx).T


# speedup vs baseline: 1.1359x; 1.1359x over previous
import functools

import jax
import jax.numpy as jnp
from jax import lax
from jax.experimental import pallas as pl
from jax.experimental.pallas import tpu as pltpu
from jax.experimental.pallas import tpu_sc as plsc

NUM_CORES = 2
NUM_SUBCORES = 16
NUM_WORKERS = NUM_CORES * NUM_SUBCORES
LANES = 16
CHUNK = 128
NCHUNK = 4


@jax.jit
def kernel(hyperparameters, table):
    num_emb, em_size = table.shape
    batch = hyperparameters.shape[0]
    idx = jnp.squeeze(hyperparameters, axis=1).astype(jnp.int32)
    padded = jnp.pad(table, ((0, 0), (0, 128 - em_size)))
    b_per_w = batch // NUM_WORKERS
    assert b_per_w == CHUNK * NCHUNK

    mesh = plsc.VectorSubcoreMesh(core_axis_name="c", subcore_axis_name="s")

    @functools.partial(
        pl.kernel,
        mesh=mesh,
        out_type=jax.ShapeDtypeStruct((batch, em_size), jnp.float32),
        scratch_types=[
            pltpu.VMEM((b_per_w,), jnp.int32),
            pltpu.VMEM((b_per_w, 128), jnp.float32),
            pltpu.VMEM((CHUNK, em_size), jnp.float32),
            pltpu.VMEM((CHUNK, em_size), jnp.float32),
            pltpu.SemaphoreType.DMA,
            pltpu.SemaphoreType.DMA,
            pltpu.SemaphoreType.DMA,
            pltpu.SemaphoreType.DMA,
            pltpu.SemaphoreType.DMA,
        ],
    )
    def emb_lookup(table_hbm, idx_hbm, out_hbm, idx_v, rows_v, out_v0, out_v1,
                   sem0, sem1, sem2, sem3, store_sem):
        wid = lax.axis_index("s") * NUM_CORES + lax.axis_index("c")
        base = wid * b_per_w
        gather_sems = [sem0, sem1, sem2, sem3]
        out_bufs = [out_v0, out_v1]
        pltpu.sync_copy(idx_hbm.at[pl.ds(base, b_per_w)], idx_v)

        for chunk in range(NCHUNK):
            c0 = chunk * CHUNK

            @pl.loop(0, CHUNK, step=LANES)
            def _(k, c0=c0, sem=gather_sems[chunk]):
                v = idx_v[pl.ds(c0 + k, LANES)]
                pltpu.async_copy(
                    table_hbm.at[v], rows_v.at[pl.ds(c0 + k, LANES)], sem
                )

        for chunk in range(NCHUNK):
            c0 = chunk * CHUNK
            out_v = out_bufs[chunk % 2]
            pltpu.make_async_copy(
                table_hbm.at[pl.ds(0, CHUNK)],
                rows_v.at[pl.ds(c0, CHUNK)],
                gather_sems[chunk],
            ).wait()
            if chunk >= 2:
                pltpu.make_async_copy(
                    out_v, out_hbm.at[pl.ds(0, CHUNK)], store_sem
                ).wait()

            @pl.loop(0, CHUNK, step=1)
            def _(r, c0=c0, out_v=out_v):
                for c in range(0, em_size, LANES):
                    out_v[r, pl.ds(c, LANES)] = rows_v[c0 + r, pl.ds(c, LANES)]

            pltpu.async_copy(
                out_v, out_hbm.at[pl.ds(base + c0, CHUNK)], store_sem
            )

        for _ in range(2):
            pltpu.make_async_copy(
                out_v0, out_hbm.at[pl.ds(0, CHUNK)], store_sem
            ).wait()

    return emb_lookup(padded, idx)


# speedup vs baseline: 1.1437x; 1.0069x over previous
import functools

import jax
import jax.numpy as jnp
from jax import lax
from jax.experimental import pallas as pl
from jax.experimental.pallas import tpu as pltpu
from jax.experimental.pallas import tpu_sc as plsc

NUM_CORES = 2
NUM_SUBCORES = 16
NUM_WORKERS = NUM_CORES * NUM_SUBCORES
LANES = 16
CHUNK = 128
NCHUNK = 4


@jax.jit
def kernel(hyperparameters, table):
    num_emb, em_size = table.shape
    batch = hyperparameters.shape[0]
    idx = jnp.squeeze(hyperparameters, axis=1).astype(jnp.int32)
    padded = jnp.pad(table, ((0, 0), (0, 128 - em_size)))
    b_per_w = batch // NUM_WORKERS
    assert b_per_w == CHUNK * NCHUNK

    mesh = plsc.VectorSubcoreMesh(core_axis_name="c", subcore_axis_name="s")

    @functools.partial(
        pl.kernel,
        mesh=mesh,
        out_type=jax.ShapeDtypeStruct((batch, 128), jnp.float32),
        scratch_types=[
            pltpu.VMEM((b_per_w,), jnp.int32),
            pltpu.VMEM((b_per_w, 128), jnp.float32),
            pltpu.SemaphoreType.DMA,
            pltpu.SemaphoreType.DMA,
            pltpu.SemaphoreType.DMA,
            pltpu.SemaphoreType.DMA,
            pltpu.SemaphoreType.DMA,
        ],
    )
    def emb_lookup(table_hbm, idx_hbm, out_hbm, idx_v, rows_v,
                   sem0, sem1, sem2, sem3, store_sem):
        wid = lax.axis_index("s") * NUM_CORES + lax.axis_index("c")
        base = wid * b_per_w
        gather_sems = [sem0, sem1, sem2, sem3]
        pltpu.sync_copy(idx_hbm.at[pl.ds(base, b_per_w)], idx_v)

        for chunk in range(NCHUNK):
            c0 = chunk * CHUNK

            @pl.loop(0, CHUNK, step=LANES)
            def _(k, c0=c0, sem=gather_sems[chunk]):
                v = idx_v[pl.ds(c0 + k, LANES)]
                pltpu.async_copy(
                    table_hbm.at[v], rows_v.at[pl.ds(c0 + k, LANES)], sem
                )

        for chunk in range(NCHUNK):
            c0 = chunk * CHUNK
            pltpu.make_async_copy(
                table_hbm.at[pl.ds(0, CHUNK)],
                rows_v.at[pl.ds(c0, CHUNK)],
                gather_sems[chunk],
            ).wait()
            pltpu.async_copy(
                rows_v.at[pl.ds(c0, CHUNK)],
                out_hbm.at[pl.ds(base + c0, CHUNK)],
                store_sem,
            )

        pltpu.make_async_copy(rows_v, out_hbm.at[pl.ds(0, b_per_w)],
                              store_sem).wait()

    return emb_lookup(padded, idx)[:, :em_size]
